# baseline (speedup 1.0000x reference)
import functools

import jax
import jax.numpy as jnp
from jax import lax
from jax.experimental import pallas as pl
from jax.experimental.pallas import tpu as pltpu
from jax.experimental.pallas import tpu_sc as plsc

_CHUNK = 40
_LANES = 16
_NBUF = 4
_ZBLK = 40


def _out_body(p_ref, w_ref, o_ref):
    h = p_ref[0] + p_ref[1]
    t = jnp.dot(h, w_ref[...], preferred_element_type=jnp.float32)
    o_ref[...] = jnp.where(t >= 0.0, t, 0.2 * t)


@functools.cache
def _sc_spmm(n_nodes, n_edges, feat, nc, ns):
    nw = nc * ns
    epw = n_edges // nw
    assert n_edges % nw == 0 and epw % _CHUNK == 0
    n_chunks = epw // _CHUNK
    n_main = (n_chunks // _NBUF) * _NBUF
    n_tail = n_chunks - n_main
    assert n_main >= 2 * _NBUF
    assert n_nodes % _ZBLK == 0
    n_blocks = n_nodes // _ZBLK
    blocks_per_tile = -(-n_blocks // ns)
    n_vec = feat // _LANES
    groups = [(0, 0), (16, 0), (24, 8)]

    mesh = plsc.VectorSubcoreMesh(core_axis_name="c", subcore_axis_name="s")

    @functools.partial(
        pl.kernel,
        mesh=mesh,
        out_type=jax.ShapeDtypeStruct((nc, n_nodes, feat), jnp.float32),
        scratch_types=(
            [
                pltpu.VMEM((epw,), jnp.int32),
                pltpu.VMEM((epw,), jnp.int32),
                pltpu.VMEM((epw,), jnp.float32),
            ]
            + [pltpu.VMEM((_CHUNK, feat), jnp.float32)] * _NBUF
            + [pltpu.VMEM((_CHUNK,), jnp.int32)] * 2
            + [pltpu.VMEM_SHARED((n_nodes, feat), jnp.float32)]
            + [pltpu.SemaphoreType.DMA] * (1 + _NBUF + 2)
        ),
    )
    def spmm(xfeat, srcs, dsts, ew, out, src_v, dst_v, w_v, *scr):
        rows_v = scr[:_NBUF]
        sdst = scr[_NBUF:_NBUF + 2]
        acc = scr[_NBUF + 2]
        sem_pre = scr[_NBUF + 3]
        sem_ga = scr[_NBUF + 4:_NBUF + 4 + _NBUF]
        sem_sc = scr[_NBUF + 4 + _NBUF:]

        c = lax.axis_index("c")
        s = lax.axis_index("s")
        wid = s * nc + c
        base0 = wid * epw

        pltpu.async_copy(srcs.at[pl.ds(base0, epw)], src_v, sem_pre)
        pltpu.async_copy(dsts.at[pl.ds(base0, epw)], dst_v, sem_pre)
        pltpu.async_copy(ew.at[pl.ds(base0, epw)], w_v, sem_pre)

        def zero_rows(e, carry):
            for j in range(n_vec):
                scr[0][e, pl.ds(j * _LANES, _LANES)] = (
                    jnp.zeros((_LANES,), jnp.float32))
            return carry
        lax.fori_loop(0, _ZBLK, zero_rows, 0)

        for i in range(blocks_per_tile):
            blk = s + i * ns

            @pl.when(blk < n_blocks)
            def _():
                pltpu.sync_copy(scr[0], acc.at[pl.ds(blk * _ZBLK, _ZBLK)])

        pltpu.make_async_copy(srcs.at[pl.ds(base0, epw)], src_v, sem_pre).wait()
        pltpu.make_async_copy(dsts.at[pl.ds(base0, epw)], dst_v, sem_pre).wait()
        pltpu.make_async_copy(ew.at[pl.ds(base0, epw)], w_v, sem_pre).wait()
        plsc.subcore_barrier()

        def start_gather(k, b):
            pltpu.async_copy(
                xfeat.at[src_v.at[pl.ds(k * _CHUNK, _CHUNK)]],
                rows_v[b], sem_ga[b])

        def wait_gather(b):
            pltpu.make_async_copy(
                xfeat.at[src_v.at[pl.ds(0, _CHUNK)]], rows_v[b],
                sem_ga[b]).wait()

        def start_scatter(b, p):
            pltpu.async_copy(rows_v[b], acc.at[sdst[p]], sem_sc[p], add=True)

        def wait_scatter(b, p):
            pltpu.make_async_copy(rows_v[b], acc.at[sdst[p]], sem_sc[p]).wait()

        def compute(k, b, p):
            koff = k * _CHUNK
            for goff, g0 in groups:
                wv = w_v[pl.ds(koff + goff, _LANES)]
                for e2 in range(g0, _LANES):
                    e = goff + e2
                    w = wv[e2]
                    for j in range(n_vec):
                        sl = pl.ds(j * _LANES, _LANES)
                        rows_v[b][e, sl] = rows_v[b][e, sl] * w
            for goff, _ in groups:
                sdst[p][pl.ds(goff, _LANES)] = dst_v[pl.ds(koff + goff, _LANES)]

        start_gather(0, 0)
        start_gather(1, 1)

        def guard(cond, fn):
            if isinstance(cond, bool):
                if cond:
                    fn()
            else:
                pl.when(cond)(fn)

        def body(k, b, parity):
            guard(k >= 2, lambda: wait_scatter((b + 2) % _NBUF, parity))
            guard(k + 2 < n_chunks,
                  lambda: start_gather(k + 2, (b + 2) % _NBUF))
            wait_gather(b)
            compute(k, b, parity)
            start_scatter(b, parity)

        def outer_body(o, carry):
            k0 = o * _NBUF
            for b in range(_NBUF):
                body(k0 + b, b, b % 2)
            return carry
        lax.fori_loop(0, n_main // _NBUF, outer_body, 0)

        for t in range(n_tail):
            k = n_main + t
            body(k, k % _NBUF, (k % _NBUF) % 2)

        wait_scatter((n_chunks - 2) % _NBUF, (n_chunks - 2) % 2)
        wait_scatter((n_chunks - 1) % _NBUF, (n_chunks - 1) % 2)

        plsc.subcore_barrier()

        for i in range(blocks_per_tile):
            blk = s + i * ns

            @pl.when(blk < n_blocks)
            def _():
                sl = pl.ds(blk * _ZBLK, _ZBLK)
                pltpu.sync_copy(acc.at[sl], out.at[c, sl])

    return spmm


def kernel(features, edge_index, edge_weight, weight):
    n, f_in = features.shape
    f_out = weight.shape[1]
    e = edge_weight.shape[0]

    info = plsc.get_sparse_core_info()
    partials = _sc_spmm(n, e, f_in, info.num_cores, info.num_subcores)(
        features, edge_index[0], edge_index[1], edge_weight)

    bm = 1000
    out = pl.pallas_call(
        _out_body,
        grid=(n // bm,),
        in_specs=[
            pl.BlockSpec((2, bm, f_in), lambda i: (0, i, 0)),
            pl.BlockSpec((f_in, f_out), lambda i: (0, 0)),
        ],
        out_specs=pl.BlockSpec((bm, f_out), lambda i: (i, 0)),
        out_shape=jax.ShapeDtypeStruct((n, f_out), jnp.float32),
    )(partials, weight)
    return out
